# baseline (speedup 1.0000x reference)
import functools

import jax
import jax.numpy as jnp
from jax import lax
from jax.experimental import pallas as pl
from jax.experimental.pallas import tpu as pltpu, tpu_sc as plsc

VOCAB = 23236
DIM = 1728
DIMP = 1792
BATCH = 1024
SEQ = 50
NROWS = BATCH * SEQ

NC = 2
NS = 16
NW = NC * NS
ROWS_PER_W = NROWS // NW

C = 16
NBUF = 4
NCHUNK = ROWS_PER_W // C
NGROUP = NCHUNK // NBUF


def _glyph_gather(idx_hbm, table_hbm, out_hbm, idx_v, *rest):
    bufs = rest[:NBUF]
    gsems = rest[NBUF : 2 * NBUF]
    wsems = rest[2 * NBUF : 3 * NBUF]

    wid = lax.axis_index("s") * NC + lax.axis_index("c")
    base = wid * ROWS_PER_W

    pltpu.sync_copy(idx_hbm.at[wid], idx_v)

    for b in range(NBUF):
        pltpu.async_copy(table_hbm.at[idx_v.at[b]], bufs[b], gsems[b])

    def group(g, carry):
        for b in range(NBUF):
            j = g * NBUF + b
            row0 = base + j * C
            pltpu.make_async_copy(
                table_hbm.at[idx_v.at[j]], bufs[b], gsems[b]
            ).wait()
            pltpu.async_copy(
                bufs[b], out_hbm.at[pl.ds(row0, C), pl.ds(0, DIM)], wsems[b]
            )
            pltpu.make_async_copy(
                bufs[b], out_hbm.at[pl.ds(row0, C), pl.ds(0, DIM)], wsems[b]
            ).wait()
            pltpu.async_copy(
                table_hbm.at[idx_v.at[j + NBUF]], bufs[b], gsems[b]
            )
        return carry

    lax.fori_loop(0, NGROUP - 1, group, 0, unroll=False)

    for b in range(NBUF):
        j = (NGROUP - 1) * NBUF + b
        row0 = base + j * C
        pltpu.make_async_copy(
            table_hbm.at[idx_v.at[j]], bufs[b], gsems[b]
        ).wait()
        pltpu.async_copy(
            bufs[b], out_hbm.at[pl.ds(row0, C), pl.ds(0, DIM)], wsems[b]
        )
    for b in range(NBUF):
        j = (NGROUP - 1) * NBUF + b
        row0 = base + j * C
        pltpu.make_async_copy(
            bufs[b], out_hbm.at[pl.ds(row0, C), pl.ds(0, DIM)], wsems[b]
        ).wait()


def _sc_gather(idx3, table):
    mesh = plsc.VectorSubcoreMesh(core_axis_name="c", subcore_axis_name="s")
    scratch = (
        [pltpu.VMEM((NCHUNK, C), jnp.int32)]
        + [pltpu.VMEM((C, DIM), jnp.float32) for _ in range(NBUF)]
        + [pltpu.SemaphoreType.DMA for _ in range(2 * NBUF)]
    )
    fn = pl.kernel(
        _glyph_gather,
        out_type=jax.ShapeDtypeStruct((NROWS, DIMP), jnp.float32),
        mesh=mesh,
        scratch_types=scratch,
        compiler_params=pltpu.CompilerParams(use_tc_tiling_on_sc=False),
    )
    return fn(idx3, table)


def _tc_transpose_body(x_ref, o_ref):
    o_ref[0] = jnp.transpose(x_ref[0, :, :DIM], (1, 0))


def _tc_transpose(x3):
    return pl.pallas_call(
        _tc_transpose_body,
        grid=(SEQ,),
        in_specs=[pl.BlockSpec((1, BATCH, DIMP), lambda s: (s, 0, 0))],
        out_specs=pl.BlockSpec((1, DIM, BATCH), lambda s: (s, 0, 0)),
        out_shape=jax.ShapeDtypeStruct((SEQ, DIM, BATCH), jnp.float32),
    )(x3)


@jax.jit
def _run(input_ids, table):
    idx3 = input_ids.T.astype(jnp.int32).reshape(NW, NCHUNK, C)
    out2 = _sc_gather(idx3, table)
    oT = _tc_transpose(out2.reshape(SEQ, BATCH, DIMP))
    return jnp.transpose(oT, (2, 0, 1))


def kernel(input_ids, table):
    return _run(input_ids, table)


# speedup vs baseline: 1.5179x; 1.5179x over previous
import jax
import jax.numpy as jnp
from jax import lax
from jax.experimental import pallas as pl
from jax.experimental.pallas import tpu as pltpu, tpu_sc as plsc

VOCAB = 23236
DIM = 1728
BATCH = 1024
SEQ = 50
NROWS = BATCH * SEQ

NC = 2
NS = 16
NW = NC * NS
ROWS_PER_W = NROWS // NW

C = 16
NBUF = 4
NCHUNK = ROWS_PER_W // C
NGROUP = NCHUNK // NBUF

TBLK = 512


def _glyph_gather(idx_hbm, table_hbm, out_hbm, idx_v, *rest):
    bufs = rest[:NBUF]
    gsems = rest[NBUF : 2 * NBUF]
    wsems = rest[2 * NBUF : 3 * NBUF]

    wid = lax.axis_index("s") * NC + lax.axis_index("c")
    base = wid * ROWS_PER_W

    pltpu.sync_copy(idx_hbm.at[wid], idx_v)

    for b in range(NBUF):
        pltpu.async_copy(table_hbm.at[idx_v.at[b]], bufs[b], gsems[b])

    def group(g, carry):
        for b in range(NBUF):
            j = g * NBUF + b
            row0 = base + j * C
            pltpu.make_async_copy(
                table_hbm.at[idx_v.at[j]], bufs[b], gsems[b]
            ).wait()
            pltpu.async_copy(bufs[b], out_hbm.at[pl.ds(row0, C)], wsems[b])
            pltpu.make_async_copy(
                bufs[b], out_hbm.at[pl.ds(row0, C)], wsems[b]
            ).wait()
            pltpu.async_copy(
                table_hbm.at[idx_v.at[j + NBUF]], bufs[b], gsems[b]
            )
        return carry

    lax.fori_loop(0, NGROUP - 1, group, 0, unroll=False)

    for b in range(NBUF):
        j = (NGROUP - 1) * NBUF + b
        row0 = base + j * C
        pltpu.make_async_copy(
            table_hbm.at[idx_v.at[j]], bufs[b], gsems[b]
        ).wait()
        pltpu.async_copy(bufs[b], out_hbm.at[pl.ds(row0, C)], wsems[b])
    for b in range(NBUF):
        j = (NGROUP - 1) * NBUF + b
        row0 = base + j * C
        pltpu.make_async_copy(
            bufs[b], out_hbm.at[pl.ds(row0, C)], wsems[b]
        ).wait()


def _sc_gather(idx3, table):
    mesh = plsc.VectorSubcoreMesh(core_axis_name="c", subcore_axis_name="s")
    scratch = (
        [pltpu.VMEM((NCHUNK, C), jnp.int32)]
        + [pltpu.VMEM((C, DIM), jnp.float32) for _ in range(NBUF)]
        + [pltpu.SemaphoreType.DMA for _ in range(2 * NBUF)]
    )
    fn = pl.kernel(
        _glyph_gather,
        out_type=jax.ShapeDtypeStruct((NROWS, DIM), jnp.float32),
        mesh=mesh,
        scratch_types=scratch,
        compiler_params=pltpu.CompilerParams(use_tc_tiling_on_sc=False),
    )
    return fn(idx3, table)


def _tc_transpose_body(x_ref, o_ref):
    o_ref[...] = jnp.transpose(x_ref[...], (1, 0))


def _tc_table_transpose(tT):
    grid = (VOCAB + TBLK - 1) // TBLK
    return pl.pallas_call(
        _tc_transpose_body,
        grid=(grid,),
        in_specs=[pl.BlockSpec((DIM, TBLK), lambda i: (0, i))],
        out_specs=pl.BlockSpec((TBLK, DIM), lambda i: (i, 0)),
        out_shape=jax.ShapeDtypeStruct((VOCAB, DIM), jnp.float32),
    )(tT)


@jax.jit
def _run(input_ids, table):
    tR = _tc_table_transpose(table.T)
    idx3 = input_ids.astype(jnp.int32).reshape(NW, NCHUNK, C)
    out = _sc_gather(idx3, tR)
    return out.reshape(BATCH, SEQ, DIM)


def kernel(input_ids, table):
    return _run(input_ids, table)
